# baseline (speedup 1.0000x reference)
import functools

import jax
import jax.numpy as jnp
from jax import lax
from jax.experimental import pallas as pl
from jax.experimental.pallas import tpu as pltpu
from jax.experimental.pallas import tpu_sc as plsc

D_MODEL = 512
NUM_TILES = 16
TOP_K = 4
B = 4096
TERNARY_W = 0.01
SPARSITY_W = 0.005

TBLK_A = 128
TBLK_B = 512
RBLK = 256
NASSIGN = B * TOP_K
NB = (NASSIGN + NUM_TILES * (RBLK - 1)) // RBLK + 1
NPAD = NB * RBLK
NEG_INF = -3.0e38


def _dot(a, b):
    return jax.lax.dot_general(a, b, (((1,), (0,)), ((), ())),
                               preferred_element_type=jnp.float32)


def _prep_body(ints_ref, op_embed_ref, w_in_ref, b_in_ref, w_r_ref, b_r_ref,
               x_ref, topi_ref, topn_ref, lr_ref, il_ref, run_ref):
    i = pl.program_id(0)
    ints = ints_ref[...]
    op = ints[:, 0:1]
    a = ints[:, 1:2]
    b = ints[:, 2:3]
    c = ints[:, 3:4]
    T = ints.shape[0]

    @pl.when(i == 0)
    def _():
        il_ref[...] = jnp.zeros_like(il_ref)
        run_ref[...] = jnp.zeros_like(run_ref)

    op_emb = jnp.zeros((T, 16), jnp.float32)
    for j in range(8):
        m = (op == j).astype(jnp.float32)
        op_emb = op_emb + m * op_embed_ref[j:j + 1, :]

    bit_iota = lax.broadcasted_iota(jnp.int32, (1, 8), 1)
    a_bits = ((lax.shift_right_logical(a, bit_iota)) & 1).astype(jnp.float32)
    b_bits = ((lax.shift_right_logical(b, bit_iota)) & 1).astype(jnp.float32)
    c_f = c.astype(jnp.float32)

    feats = jnp.concatenate(
        [op_emb, a_bits, b_bits, c_f, jnp.zeros((T, 128 - 33), jnp.float32)],
        axis=1)

    x = _dot(feats, w_in_ref[...]) + b_in_ref[...]
    x_ref[...] = x

    logits = _dot(x, w_r_ref[...]) + b_r_ref[...]
    m = jnp.max(logits, axis=1, keepdims=True)
    e = jnp.exp(logits - m)
    gates = e / jnp.sum(e, axis=1, keepdims=True)

    iota16 = lax.broadcasted_iota(jnp.int32, (T, NUM_TILES), 1)
    v = gates
    tis, tvs = [], []
    for _ in range(TOP_K):
        mx = jnp.max(v, axis=1, keepdims=True)
        idx = jnp.min(jnp.where(v == mx, iota16, NUM_TILES), axis=1,
                      keepdims=True)
        tis.append(idx)
        tvs.append(mx)
        v = jnp.where(iota16 == idx, NEG_INF, v)
    topi = jnp.concatenate(tis, axis=1)
    topv = jnp.concatenate(tvs, axis=1)
    topn = topv / jnp.sum(topv, axis=1, keepdims=True)
    topi_ref[...] = topi
    topn_ref[...] = topn

    ohk = [(iota16 == topi[:, k:k + 1]).astype(jnp.float32)
           for k in range(TOP_K)]
    disp = ohk[0] + ohk[1] + ohk[2] + ohk[3]

    tri = (lax.broadcasted_iota(jnp.int32, (T, T), 0) >
           lax.broadcasted_iota(jnp.int32, (T, T), 1)).astype(jnp.float32)
    ct = _dot(tri, disp)
    base = ct + run_ref[0:1, 0:16]
    prev = jnp.zeros((T, NUM_TILES), jnp.float32)
    lrs = []
    for k in range(TOP_K):
        lrs.append(jnp.sum(ohk[k] * (base + prev), axis=1, keepdims=True))
        prev = prev + ohk[k]
    lr_ref[...] = jnp.concatenate(lrs, axis=1).astype(jnp.int32)
    run_ref[0:1, 0:16] += jnp.sum(disp, axis=0, keepdims=True)

    ones_col = jnp.ones((T, 1), jnp.float32)
    impcol = jax.lax.dot_general(gates, ones_col, (((0,), (0,)), ((), ())),
                                 preferred_element_type=jnp.float32)
    cntcol = jax.lax.dot_general(disp, ones_col, (((0,), (0,)), ((), ())),
                                 preferred_element_type=jnp.float32)
    il_ref[0:NUM_TILES, 0:1] += impcol
    il_ref[0:NUM_TILES, 1:2] += cntcol


def _prep_call(ints, op_embed, w_in_p, b_in, w_r, b_r):
    nblk = B // TBLK_A
    return pl.pallas_call(
        _prep_body,
        grid=(nblk,),
        in_specs=[
            pl.BlockSpec((TBLK_A, 4), lambda i: (i, 0)),
            pl.BlockSpec((8, 16), lambda i: (0, 0)),
            pl.BlockSpec((128, D_MODEL), lambda i: (0, 0)),
            pl.BlockSpec((1, D_MODEL), lambda i: (0, 0)),
            pl.BlockSpec((D_MODEL, NUM_TILES), lambda i: (0, 0)),
            pl.BlockSpec((1, NUM_TILES), lambda i: (0, 0)),
        ],
        out_specs=[
            pl.BlockSpec((TBLK_A, D_MODEL), lambda i: (i, 0)),
            pl.BlockSpec((TBLK_A, TOP_K), lambda i: (i, 0)),
            pl.BlockSpec((TBLK_A, TOP_K), lambda i: (i, 0)),
            pl.BlockSpec((TBLK_A, TOP_K), lambda i: (i, 0)),
            pl.BlockSpec((NUM_TILES, 128), lambda i: (0, 0)),
        ],
        out_shape=[
            jax.ShapeDtypeStruct((B, D_MODEL), jnp.float32),
            jax.ShapeDtypeStruct((B, TOP_K), jnp.int32),
            jax.ShapeDtypeStruct((B, TOP_K), jnp.float32),
            jax.ShapeDtypeStruct((B, TOP_K), jnp.int32),
            jax.ShapeDtypeStruct((NUM_TILES, 128), jnp.float32),
        ],
        scratch_shapes=[pltpu.VMEM((8, 128), jnp.float32)],
    )(ints, op_embed, w_in_p, b_in, w_r, b_r)


def _plan_body(topi_ref, lr_ref, topn_ref, il_ref, p_ref, plan_ref,
               grows_ref):
    i = pl.program_id(0)
    cnt = il_ref[0:NUM_TILES, 1:2].astype(jnp.int32)
    cnt_pad = ((cnt + (RBLK - 1)) // RBLK) * RBLK
    tri16 = (lax.broadcasted_iota(jnp.int32, (NUM_TILES, NUM_TILES), 0) >
             lax.broadcasted_iota(jnp.int32, (NUM_TILES, NUM_TILES), 1)
             ).astype(jnp.float32)
    off_col = jax.lax.dot_general(
        tri16, cnt_pad.astype(jnp.float32), (((1,), (0,)), ((), ())),
        preferred_element_type=jnp.float32).astype(jnp.int32)

    @pl.when(i == 0)
    def _():
        eye16 = (lax.broadcasted_iota(jnp.int32, (NUM_TILES, NUM_TILES), 0) ==
                 lax.broadcasted_iota(jnp.int32, (NUM_TILES, NUM_TILES), 1)
                 ).astype(jnp.float32)
        off_row = jax.lax.dot_general(
            off_col.astype(jnp.float32), eye16, (((0,), (0,)), ((), ())),
            preferred_element_type=jnp.float32)
        thr = (lax.broadcasted_iota(jnp.int32, (NB, 1), 0) * RBLK
               ).astype(jnp.float32)
        cmp = (off_row <= thr).astype(jnp.float32)
        be_col = jax.lax.dot_general(
            cmp, jnp.ones((NUM_TILES, 1), jnp.float32),
            (((1,), (0,)), ((), ())),
            preferred_element_type=jnp.float32).astype(jnp.int32) - 1
        plan_ref[...] = jnp.zeros_like(plan_ref)
        plan_ref[:, 0:1] = be_col
        na = jnp.sum(cnt_pad) // RBLK
        plan_ref[0:1, 1:2] = jnp.reshape(na, (1, 1))
        plan_ref[0:NUM_TILES, 2:3] = off_col

    topi = topi_ref[...]
    offsel = jnp.zeros(topi.shape, jnp.int32)
    for e in range(NUM_TILES):
        offsel = offsel + jnp.where(topi == e, off_col[e, 0], 0)
    p_ref[...] = offsel + lr_ref[...]

    T = topi.shape[0]
    R = T * TOP_K
    rep = (lax.broadcasted_iota(jnp.int32, (R, T), 0) // TOP_K ==
           lax.broadcasted_iota(jnp.int32, (R, T), 1)).astype(jnp.float32)
    crep = _dot(rep, topn_ref[...])
    kmod = lax.broadcasted_iota(jnp.int32, (R, 1), 0) % TOP_K
    gcol = jnp.zeros((R, 1), jnp.float32)
    for k in range(TOP_K):
        gcol = gcol + jnp.where(kmod == k, crep[:, k:k + 1], 0.0)
    grows_ref[...] = jnp.broadcast_to(gcol, (R, 16))


def _plan_call(topi, lr, topn, il):
    nblk = B // TBLK_A
    return pl.pallas_call(
        _plan_body,
        grid=(nblk,),
        in_specs=[
            pl.BlockSpec((TBLK_A, TOP_K), lambda i: (i, 0)),
            pl.BlockSpec((TBLK_A, TOP_K), lambda i: (i, 0)),
            pl.BlockSpec((TBLK_A, TOP_K), lambda i: (i, 0)),
            pl.BlockSpec((NUM_TILES, 128), lambda i: (0, 0)),
        ],
        out_specs=[
            pl.BlockSpec((TBLK_A, TOP_K), lambda i: (i, 0)),
            pl.BlockSpec((NB, 128), lambda i: (0, 0)),
            pl.BlockSpec((TBLK_A * TOP_K, 16), lambda i: (i, 0)),
        ],
        out_shape=[
            jax.ShapeDtypeStruct((B, TOP_K), jnp.int32),
            jax.ShapeDtypeStruct((NB, 128), jnp.int32),
            jax.ShapeDtypeStruct((NASSIGN, 16), jnp.float32),
        ],
    )(topi, lr, topn, il)


SC_CHUNK = 64


def _make_scatter():
    info = plsc.get_sparse_core_info()
    nw = info.num_cores * info.num_subcores
    rows_per_w = NASSIGN // nw
    nchunk = rows_per_w // SC_CHUNK
    mesh = plsc.VectorSubcoreMesh(core_axis_name="c", subcore_axis_name="s")

    @functools.partial(
        pl.kernel, mesh=mesh,
        out_type=jax.ShapeDtypeStruct((NPAD, D_MODEL), jnp.float32),
        scratch_types=[
            pltpu.VMEM((SC_CHUNK,), jnp.int32),
            pltpu.VMEM((SC_CHUNK,), jnp.int32),
            pltpu.VMEM((SC_CHUNK, D_MODEL), jnp.float32),
            pltpu.SemaphoreType.DMA,
            pltpu.SemaphoreType.DMA,
        ],
    )
    def scatter_k(x_hbm, p_hbm, xs_hbm, tok_v, pidx_v, rows_v, sem1, sem2):
        wid = lax.axis_index("s") * info.num_cores + lax.axis_index("c")
        base = wid * rows_per_w
        for ci in range(nchunk):
            cb = base + ci * SC_CHUNK
            for q in range(SC_CHUNK // 16):
                tok_v[pl.ds(q * 16, 16)] = (
                    cb + q * 16 + lax.iota(jnp.int32, 16)) >> 2
            pltpu.sync_copy(p_hbm.at[pl.ds(cb, SC_CHUNK)], pidx_v)
            pltpu.async_copy(x_hbm.at[tok_v], rows_v, sem1).wait()
            pltpu.async_copy(rows_v, xs_hbm.at[pidx_v], sem2).wait()

    return scatter_k


def _ffn_body(be_ref, na_ref, xs_ref, w1_ref, b1_ref, w2_ref, b2_ref,
              ys_ref):
    i = pl.program_id(0)

    @pl.when(i < na_ref[0])
    def _():
        h = jnp.maximum(_dot(xs_ref[...], w1_ref[0]) + b1_ref[0], 0.0)
        ys_ref[...] = _dot(h, w2_ref[0]) + b2_ref[0]


def _ffn_call(be, na, xs, w1, b1, w2, b2):
    grid_spec = pltpu.PrefetchScalarGridSpec(
        num_scalar_prefetch=2,
        grid=(NB,),
        in_specs=[
            pl.BlockSpec((RBLK, D_MODEL), lambda i, be, na: (i, 0)),
            pl.BlockSpec((1, D_MODEL, D_MODEL), lambda i, be, na: (be[i], 0, 0)),
            pl.BlockSpec((1, 1, D_MODEL), lambda i, be, na: (be[i], 0, 0)),
            pl.BlockSpec((1, D_MODEL, D_MODEL), lambda i, be, na: (be[i], 0, 0)),
            pl.BlockSpec((1, 1, D_MODEL), lambda i, be, na: (be[i], 0, 0)),
        ],
        out_specs=pl.BlockSpec((RBLK, D_MODEL), lambda i, be, na: (i, 0)),
    )
    return pl.pallas_call(
        _ffn_body,
        grid_spec=grid_spec,
        out_shape=jax.ShapeDtypeStruct((NPAD, D_MODEL), jnp.float32),
    )(be, na, xs, w1, b1, w2, b2)


CB_TOK = 8


def _make_combine():
    info = plsc.get_sparse_core_info()
    nw = info.num_cores * info.num_subcores
    tok_per_w = B // nw
    nchunk = tok_per_w // CB_TOK
    crows = CB_TOK * TOP_K
    mesh = plsc.VectorSubcoreMesh(core_axis_name="c", subcore_axis_name="s")

    @functools.partial(
        pl.kernel, mesh=mesh,
        out_type=jax.ShapeDtypeStruct((B, D_MODEL), jnp.float32),
        scratch_types=[
            pltpu.VMEM((crows,), jnp.int32),
            pltpu.VMEM((crows, 16), jnp.float32),
            pltpu.VMEM((crows, D_MODEL), jnp.float32),
            pltpu.VMEM((CB_TOK, D_MODEL), jnp.float32),
            pltpu.SemaphoreType.DMA,
        ],
    )
    def combine_k(ys_hbm, p_hbm, g_hbm, out_hbm, pidx_v, g_v, rows_v,
                  out_v, sem):
        wid = lax.axis_index("s") * info.num_cores + lax.axis_index("c")

        def chunk(ci, carry):
            tb = wid * tok_per_w + ci * CB_TOK
            rb = pl.multiple_of(tb * TOP_K, crows)
            pltpu.sync_copy(p_hbm.at[pl.ds(rb, crows)], pidx_v)
            pltpu.sync_copy(g_hbm.at[pl.ds(rb, crows)], g_v)
            pltpu.async_copy(ys_hbm.at[pidx_v], rows_v, sem).wait()
            for t in range(CB_TOK):
                gs = [g_v[TOP_K * t + k, pl.ds(0, 16)]
                      for k in range(TOP_K)]
                for d in range(D_MODEL // 16):
                    acc = jnp.zeros((16,), jnp.float32)
                    for k in range(TOP_K):
                        acc = acc + gs[k] * rows_v[TOP_K * t + k,
                                                   pl.ds(d * 16, 16)]
                    out_v[t, pl.ds(d * 16, 16)] = acc
            pltpu.sync_copy(out_v,
                            out_hbm.at[pl.ds(pl.multiple_of(tb, CB_TOK),
                                             CB_TOK)])
            return carry

        lax.fori_loop(0, nchunk, chunk, 0)

    return combine_k


def _aux_body(w1_ref, w2_ref, il_ref, out_ref):
    e = pl.program_id(0)

    @pl.when(e == 0)
    def _():
        out_ref[...] = jnp.zeros_like(out_ref)

    aw1 = jnp.abs(w1_ref[0])
    aw2 = jnp.abs(w2_ref[0])
    s = (jnp.sum(aw1 * jnp.abs(1.0 - aw1)) + jnp.sum(aw2 * jnp.abs(1.0 - aw2)))
    out_ref[0:1, 0:1] += jnp.reshape(s, (1, 1))

    @pl.when(e == NUM_TILES - 1)
    def _():
        imp = il_ref[0:NUM_TILES, 0:1] * (1.0 / B)
        load = il_ref[0:NUM_TILES, 1:2] * (1.0 / B)
        lb = NUM_TILES * jnp.sum(imp * load)
        tern = out_ref[0, 0] / (NUM_TILES * D_MODEL * D_MODEL)
        out_ref[0:1, 0:1] = jnp.reshape(
            SPARSITY_W * lb + TERNARY_W * tern, (1, 1))


def _aux_call(w1, w2, il):
    return pl.pallas_call(
        _aux_body,
        grid=(NUM_TILES,),
        in_specs=[
            pl.BlockSpec((1, D_MODEL, D_MODEL), lambda e: (e, 0, 0)),
            pl.BlockSpec((1, D_MODEL, D_MODEL), lambda e: (e, 0, 0)),
            pl.BlockSpec((NUM_TILES, 128), lambda e: (0, 0)),
        ],
        out_specs=pl.BlockSpec((8, 128), lambda e: (0, 0)),
        out_shape=jax.ShapeDtypeStruct((8, 128), jnp.float32),
    )(w1, w2, il)


def _head_body(out_ref, wh1_ref, bh1_ref, wh2_ref, bh2_ref, rb_ref):
    h = jnp.maximum(_dot(out_ref[...], wh1_ref[...]) + bh1_ref[...], 0.0)
    z = _dot(h, wh2_ref[...]) + bh2_ref[...]
    rb_ref[...] = 1.0 / (1.0 + jnp.exp(-z))


def _head_call(out, wh1_p, bh1_p, wh2_p, bh2):
    nblk = B // TBLK_B
    return pl.pallas_call(
        _head_body,
        grid=(nblk,),
        in_specs=[
            pl.BlockSpec((TBLK_B, D_MODEL), lambda i: (i, 0)),
            pl.BlockSpec((D_MODEL, 128), lambda i: (0, 0)),
            pl.BlockSpec((1, 128), lambda i: (0, 0)),
            pl.BlockSpec((128, 8), lambda i: (0, 0)),
            pl.BlockSpec((1, 8), lambda i: (0, 0)),
        ],
        out_specs=pl.BlockSpec((TBLK_B, 8), lambda i: (i, 0)),
        out_shape=jax.ShapeDtypeStruct((B, 8), jnp.float32),
    )(out, wh1_p, bh1_p, wh2_p, bh2)


def kernel(op_idx, a, b, c, op_embed, W_in, b_in, W_router, b_router,
           W1, b1, W2, b2, W_h1, b_h1, W_h2, b_h2):
    ints = jnp.stack([op_idx.astype(jnp.int32), a.astype(jnp.int32),
                      b.astype(jnp.int32), c.astype(jnp.int32)], axis=1)
    w_in_p = jnp.pad(W_in, ((0, 128 - 33), (0, 0)))
    wh1_p = jnp.pad(W_h1, ((0, 0), (0, 128 - 32)))
    bh1_p = jnp.pad(b_h1, (0, 128 - 32)).reshape(1, 128)
    wh2_p = jnp.pad(W_h2, ((0, 128 - 32), (0, 0)))

    x, topi, topn, lr, il = _prep_call(
        ints, op_embed, w_in_p, b_in.reshape(1, D_MODEL),
        W_router, b_router.reshape(1, NUM_TILES))
    p, plan, g_rows = _plan_call(topi, lr, topn, il)

    p_flat = p.reshape(NASSIGN)
    xs = _make_scatter()(x, p_flat)
    be = plan[:, 0]
    na = plan[0:1, 1].reshape(1)
    ys = _ffn_call(be, na, xs, W1, b1.reshape(NUM_TILES, 1, D_MODEL),
                   W2, b2.reshape(NUM_TILES, 1, D_MODEL))
    out = _make_combine()(ys, p_flat, g_rows)

    auxm = _aux_call(W1, W2, il)
    aux = auxm[0, 0]
    result_bits = _head_call(out, wh1_p, bh1_p, wh2_p, b_h2.reshape(1, 8))
    return result_bits, topi, aux


# speedup vs baseline: 1.1196x; 1.1196x over previous
import functools

import jax
import jax.numpy as jnp
from jax import lax
from jax.experimental import pallas as pl
from jax.experimental.pallas import tpu as pltpu
from jax.experimental.pallas import tpu_sc as plsc

D_MODEL = 512
NUM_TILES = 16
TOP_K = 4
B = 4096
TERNARY_W = 0.01
SPARSITY_W = 0.005

TBLK_A = 512
TBLK_B = 512
RBLK = 256
NASSIGN = B * TOP_K
NB = (NASSIGN + NUM_TILES * (RBLK - 1)) // RBLK + 1
NPAD = NB * RBLK
NEG_INF = -3.0e38


def _dot(a, b):
    return jax.lax.dot_general(a, b, (((1,), (0,)), ((), ())),
                               preferred_element_type=jnp.float32)


def _prep_body(ints_ref, opP_ref, sa_ref, sb_ref, sc_ref, w_in_ref,
               b_in_ref, w_r_ref, b_r_ref, tri_ref,
               x_ref, topi_ref, topn_ref, lr_ref, grows_ref, il_ref,
               run_ref):
    i = pl.program_id(0)
    ints = ints_ref[...]
    op = ints[:, 0:1]
    a = ints[:, 1:2]
    b = ints[:, 2:3]
    c = ints[:, 3:4]
    T = ints.shape[0]

    @pl.when(i == 0)
    def _():
        il_ref[...] = jnp.zeros_like(il_ref)
        run_ref[...] = jnp.zeros_like(run_ref)

    ones16 = jnp.ones((NUM_TILES, 1), jnp.float32)

    oh8 = (op == lax.broadcasted_iota(jnp.int32, (1, 8), 1)
           ).astype(jnp.float32)
    bit_iota = lax.broadcasted_iota(jnp.int32, (1, 8), 1)
    a_bits = ((lax.shift_right_logical(a, bit_iota)) & 1).astype(jnp.float32)
    b_bits = ((lax.shift_right_logical(b, bit_iota)) & 1).astype(jnp.float32)
    feats = (_dot(oh8, opP_ref[...]) + _dot(a_bits, sa_ref[...]) +
             _dot(b_bits, sb_ref[...]) +
             _dot(c.astype(jnp.float32), sc_ref[...]))
    x = _dot(feats, w_in_ref[...]) + b_in_ref[...]
    x_ref[...] = x

    logits = _dot(x, w_r_ref[...]) + b_r_ref[...]
    m = jnp.max(logits, axis=1, keepdims=True)
    e = jnp.exp(logits - m)
    gates = e / _dot(e, ones16)

    iota16 = lax.broadcasted_iota(jnp.int32, (T, NUM_TILES), 1)
    v = gates
    tis, tvs = [], []
    for _ in range(TOP_K):
        mx = jnp.max(v, axis=1, keepdims=True)
        idx = jnp.min(jnp.where(v == mx, iota16, NUM_TILES), axis=1,
                      keepdims=True)
        tis.append(idx)
        tvs.append(mx)
        v = jnp.where(iota16 == idx, NEG_INF, v)
    topi = jnp.concatenate(tis, axis=1)
    topv = jnp.concatenate(tvs, axis=1)
    topn = topv / _dot(topv, jnp.ones((TOP_K, 1), jnp.float32))
    topi_ref[...] = topi
    topn_ref[...] = topn
    grows_ref[...] = jnp.broadcast_to(topn[:, :, None], (T, TOP_K, 16))

    ohk = [(iota16 == topi[:, k:k + 1]).astype(jnp.float32)
           for k in range(TOP_K)]
    disp = ohk[0] + ohk[1] + ohk[2] + ohk[3]

    ct = _dot(tri_ref[...], disp)
    base = ct + run_ref[0:1, 0:16]
    prev = jnp.zeros((T, NUM_TILES), jnp.float32)
    lrs = []
    for k in range(TOP_K):
        lrs.append(_dot(ohk[k] * (base + prev), ones16))
        prev = prev + ohk[k]
    lr_ref[...] = jnp.concatenate(lrs, axis=1).astype(jnp.int32)
    run_ref[0:1, 0:16] += jnp.sum(disp, axis=0, keepdims=True)

    ones_col = jnp.ones((T, 1), jnp.float32)
    impcol = jax.lax.dot_general(gates, ones_col, (((0,), (0,)), ((), ())),
                                 preferred_element_type=jnp.float32)
    cntcol = jax.lax.dot_general(disp, ones_col, (((0,), (0,)), ((), ())),
                                 preferred_element_type=jnp.float32)
    il_ref[0:NUM_TILES, 0:1] += impcol
    il_ref[0:NUM_TILES, 1:2] += cntcol


def _prep_call(ints, opP, sa, sb, sc, w_in_p, b_in, w_r, b_r, tri):
    nblk = B // TBLK_A

    def cst(s):
        return pl.BlockSpec(s, lambda i: tuple(0 for _ in s))

    def blk(s):
        return pl.BlockSpec(s, lambda i: (i,) + tuple(0 for _ in s[1:]))

    return pl.pallas_call(
        _prep_body,
        grid=(nblk,),
        in_specs=[
            blk((TBLK_A, 4)),
            cst((8, 128)),
            cst((8, 128)),
            cst((8, 128)),
            cst((1, 128)),
            cst((128, D_MODEL)),
            cst((1, D_MODEL)),
            cst((D_MODEL, NUM_TILES)),
            cst((1, NUM_TILES)),
            cst((TBLK_A, TBLK_A)),
        ],
        out_specs=[
            blk((TBLK_A, D_MODEL)),
            blk((TBLK_A, TOP_K)),
            blk((TBLK_A, TOP_K)),
            blk((TBLK_A, TOP_K)),
            blk((TBLK_A, TOP_K, 16)),
            cst((NUM_TILES, 128)),
        ],
        out_shape=[
            jax.ShapeDtypeStruct((B, D_MODEL), jnp.float32),
            jax.ShapeDtypeStruct((B, TOP_K), jnp.int32),
            jax.ShapeDtypeStruct((B, TOP_K), jnp.float32),
            jax.ShapeDtypeStruct((B, TOP_K), jnp.int32),
            jax.ShapeDtypeStruct((B, TOP_K, 16), jnp.float32),
            jax.ShapeDtypeStruct((NUM_TILES, 128), jnp.float32),
        ],
        scratch_shapes=[pltpu.VMEM((8, 128), jnp.float32)],
    )(ints, opP, sa, sb, sc, w_in_p, b_in, w_r, b_r, tri)


def _plan_body(topi_ref, lr_ref, il_ref, p_ref, plan_ref):
    cnt = il_ref[0:NUM_TILES, 1:2].astype(jnp.int32)
    cnt_pad = ((cnt + (RBLK - 1)) // RBLK) * RBLK
    tri16 = (lax.broadcasted_iota(jnp.int32, (NUM_TILES, NUM_TILES), 0) >
             lax.broadcasted_iota(jnp.int32, (NUM_TILES, NUM_TILES), 1)
             ).astype(jnp.float32)
    off_col = jax.lax.dot_general(
        tri16, cnt_pad.astype(jnp.float32), (((1,), (0,)), ((), ())),
        preferred_element_type=jnp.float32).astype(jnp.int32)

    eye16 = (lax.broadcasted_iota(jnp.int32, (NUM_TILES, NUM_TILES), 0) ==
             lax.broadcasted_iota(jnp.int32, (NUM_TILES, NUM_TILES), 1)
             ).astype(jnp.float32)
    off_row = jax.lax.dot_general(
        off_col.astype(jnp.float32), eye16, (((0,), (0,)), ((), ())),
        preferred_element_type=jnp.float32)
    thr = (lax.broadcasted_iota(jnp.int32, (NB, 1), 0) * RBLK
           ).astype(jnp.float32)
    cmp = (off_row <= thr).astype(jnp.float32)
    be_col = jax.lax.dot_general(
        cmp, jnp.ones((NUM_TILES, 1), jnp.float32),
        (((1,), (0,)), ((), ())),
        preferred_element_type=jnp.float32).astype(jnp.int32) - 1
    plan_ref[...] = jnp.zeros_like(plan_ref)
    plan_ref[:, 0:1] = be_col
    na = jnp.sum(cnt_pad) // RBLK
    plan_ref[0:1, 1:2] = jnp.reshape(na, (1, 1))
    plan_ref[0:NUM_TILES, 2:3] = off_col

    topi = topi_ref[...]
    offsel = jnp.zeros(topi.shape, jnp.int32)
    for e in range(NUM_TILES):
        offsel = offsel + jnp.where(topi == e, off_col[e, 0], 0)
    p_ref[...] = offsel + lr_ref[...]


def _plan_call(topi, lr, il):
    return pl.pallas_call(
        _plan_body,
        grid=(1,),
        in_specs=[
            pl.BlockSpec((B, TOP_K), lambda i: (0, 0)),
            pl.BlockSpec((B, TOP_K), lambda i: (0, 0)),
            pl.BlockSpec((NUM_TILES, 128), lambda i: (0, 0)),
        ],
        out_specs=[
            pl.BlockSpec((B, TOP_K), lambda i: (0, 0)),
            pl.BlockSpec((NB, 128), lambda i: (0, 0)),
        ],
        out_shape=[
            jax.ShapeDtypeStruct((B, TOP_K), jnp.int32),
            jax.ShapeDtypeStruct((NB, 128), jnp.int32),
        ],
    )(topi, lr, il)


SC_CHUNK = 64


def _make_scatter():
    info = plsc.get_sparse_core_info()
    nw = info.num_cores * info.num_subcores
    rows_per_w = NASSIGN // nw
    nchunk = rows_per_w // SC_CHUNK
    mesh = plsc.VectorSubcoreMesh(core_axis_name="c", subcore_axis_name="s")

    @functools.partial(
        pl.kernel, mesh=mesh,
        out_type=jax.ShapeDtypeStruct((NPAD, D_MODEL), jnp.float32),
        scratch_types=[
            pltpu.VMEM((SC_CHUNK,), jnp.int32),
            pltpu.VMEM((SC_CHUNK,), jnp.int32),
            pltpu.VMEM((SC_CHUNK, D_MODEL), jnp.float32),
            pltpu.SemaphoreType.DMA,
            pltpu.SemaphoreType.DMA,
        ],
    )
    def scatter_k(x_hbm, p_hbm, xs_hbm, tok_v, pidx_v, rows_v, sem1, sem2):
        wid = lax.axis_index("s") * info.num_cores + lax.axis_index("c")
        base = wid * rows_per_w
        for ci in range(nchunk):
            cb = base + ci * SC_CHUNK
            for q in range(SC_CHUNK // 16):
                tok_v[pl.ds(q * 16, 16)] = (
                    cb + q * 16 + lax.iota(jnp.int32, 16)) >> 2
            pltpu.sync_copy(p_hbm.at[pl.ds(cb, SC_CHUNK)], pidx_v)
            pltpu.async_copy(x_hbm.at[tok_v], rows_v, sem1).wait()
            pltpu.async_copy(rows_v, xs_hbm.at[pidx_v], sem2).wait()

    return scatter_k


def _ffn_body(be_ref, na_ref, xs_ref, w1_ref, b1_ref, w2_ref, b2_ref,
              ys_ref):
    i = pl.program_id(0)

    @pl.when(i < na_ref[0])
    def _():
        h = jnp.maximum(_dot(xs_ref[...], w1_ref[0]) + b1_ref[0], 0.0)
        ys_ref[...] = _dot(h, w2_ref[0]) + b2_ref[0]


def _ffn_call(be, na, xs, w1, b1, w2, b2):
    grid_spec = pltpu.PrefetchScalarGridSpec(
        num_scalar_prefetch=2,
        grid=(NB,),
        in_specs=[
            pl.BlockSpec((RBLK, D_MODEL), lambda i, be, na: (i, 0)),
            pl.BlockSpec((1, D_MODEL, D_MODEL),
                         lambda i, be, na: (be[i], 0, 0)),
            pl.BlockSpec((1, 1, D_MODEL), lambda i, be, na: (be[i], 0, 0)),
            pl.BlockSpec((1, D_MODEL, D_MODEL),
                         lambda i, be, na: (be[i], 0, 0)),
            pl.BlockSpec((1, 1, D_MODEL), lambda i, be, na: (be[i], 0, 0)),
        ],
        out_specs=pl.BlockSpec((RBLK, D_MODEL), lambda i, be, na: (i, 0)),
    )
    return pl.pallas_call(
        _ffn_body,
        grid_spec=grid_spec,
        out_shape=jax.ShapeDtypeStruct((NPAD, D_MODEL), jnp.float32),
    )(be, na, xs, w1, b1, w2, b2)


CB_TOK = 8


def _make_combine():
    info = plsc.get_sparse_core_info()
    nw = info.num_cores * info.num_subcores
    tok_per_w = B // nw
    nchunk = tok_per_w // CB_TOK
    crows = CB_TOK * TOP_K
    mesh = plsc.VectorSubcoreMesh(core_axis_name="c", subcore_axis_name="s")

    @functools.partial(
        pl.kernel, mesh=mesh,
        out_type=jax.ShapeDtypeStruct((B, D_MODEL), jnp.float32),
        scratch_types=[
            pltpu.VMEM((crows,), jnp.int32),
            pltpu.VMEM((crows, 16), jnp.float32),
            pltpu.VMEM((crows, D_MODEL), jnp.float32),
            pltpu.VMEM((CB_TOK, D_MODEL), jnp.float32),
            pltpu.SemaphoreType.DMA,
        ],
    )
    def combine_k(ys_hbm, p_hbm, g_hbm, out_hbm, pidx_v, g_v, rows_v,
                  out_v, sem):
        wid = lax.axis_index("s") * info.num_cores + lax.axis_index("c")

        def chunk(ci, carry):
            tb = wid * tok_per_w + ci * CB_TOK
            rb = pl.multiple_of(tb * TOP_K, crows)
            pltpu.sync_copy(p_hbm.at[pl.ds(rb, crows)], pidx_v)
            pltpu.sync_copy(g_hbm.at[pl.ds(rb, crows)], g_v)
            pltpu.async_copy(ys_hbm.at[pidx_v], rows_v, sem).wait()
            for t in range(CB_TOK):
                gs = [g_v[TOP_K * t + k, pl.ds(0, 16)]
                      for k in range(TOP_K)]
                for d in range(D_MODEL // 16):
                    acc = jnp.zeros((16,), jnp.float32)
                    for k in range(TOP_K):
                        acc = acc + gs[k] * rows_v[TOP_K * t + k,
                                                   pl.ds(d * 16, 16)]
                    out_v[t, pl.ds(d * 16, 16)] = acc
            pltpu.sync_copy(out_v,
                            out_hbm.at[pl.ds(pl.multiple_of(tb, CB_TOK),
                                             CB_TOK)])
            return carry

        lax.fori_loop(0, nchunk, chunk, 0)

    return combine_k


def _aux_body(w1_ref, w2_ref, il_ref, out_ref):
    e = pl.program_id(0)

    @pl.when(e == 0)
    def _():
        out_ref[...] = jnp.zeros_like(out_ref)

    aw1 = jnp.abs(w1_ref[0])
    aw2 = jnp.abs(w2_ref[0])
    s = (jnp.sum(aw1 * jnp.abs(1.0 - aw1)) + jnp.sum(aw2 * jnp.abs(1.0 - aw2)))
    out_ref[0:1, 0:1] += jnp.reshape(s, (1, 1))

    @pl.when(e == NUM_TILES - 1)
    def _():
        imp = il_ref[0:NUM_TILES, 0:1] * (1.0 / B)
        load = il_ref[0:NUM_TILES, 1:2] * (1.0 / B)
        lb = NUM_TILES * jnp.sum(imp * load)
        tern = out_ref[0, 0] / (NUM_TILES * D_MODEL * D_MODEL)
        out_ref[0:1, 0:1] = jnp.reshape(
            SPARSITY_W * lb + TERNARY_W * tern, (1, 1))


def _aux_call(w1, w2, il):
    return pl.pallas_call(
        _aux_body,
        grid=(NUM_TILES,),
        in_specs=[
            pl.BlockSpec((1, D_MODEL, D_MODEL), lambda e: (e, 0, 0)),
            pl.BlockSpec((1, D_MODEL, D_MODEL), lambda e: (e, 0, 0)),
            pl.BlockSpec((NUM_TILES, 128), lambda e: (0, 0)),
        ],
        out_specs=pl.BlockSpec((8, 128), lambda e: (0, 0)),
        out_shape=jax.ShapeDtypeStruct((8, 128), jnp.float32),
    )(w1, w2, il)


def _head_body(out_ref, wh1_ref, bh1_ref, wh2_ref, bh2_ref, rb_ref):
    h = jnp.maximum(_dot(out_ref[...], wh1_ref[...]) + bh1_ref[...], 0.0)
    z = _dot(h, wh2_ref[...]) + bh2_ref[...]
    rb_ref[...] = 1.0 / (1.0 + jnp.exp(-z))


def _head_call(out, wh1_p, bh1_p, wh2_p, bh2):
    nblk = B // TBLK_B
    return pl.pallas_call(
        _head_body,
        grid=(nblk,),
        in_specs=[
            pl.BlockSpec((TBLK_B, D_MODEL), lambda i: (i, 0)),
            pl.BlockSpec((D_MODEL, 128), lambda i: (0, 0)),
            pl.BlockSpec((1, 128), lambda i: (0, 0)),
            pl.BlockSpec((128, 8), lambda i: (0, 0)),
            pl.BlockSpec((1, 8), lambda i: (0, 0)),
        ],
        out_specs=pl.BlockSpec((TBLK_B, 8), lambda i: (i, 0)),
        out_shape=jax.ShapeDtypeStruct((B, 8), jnp.float32),
    )(out, wh1_p, bh1_p, wh2_p, bh2)


def kernel(op_idx, a, b, c, op_embed, W_in, b_in, W_router, b_router,
           W1, b1, W2, b2, W_h1, b_h1, W_h2, b_h2):
    ints = jnp.stack([op_idx.astype(jnp.int32), a.astype(jnp.int32),
                      b.astype(jnp.int32), c.astype(jnp.int32)], axis=1)
    w_in_p = jnp.pad(W_in, ((0, 128 - 33), (0, 0)))
    opP = jnp.pad(op_embed, ((0, 0), (0, 128 - 16)))
    ar8 = jnp.arange(8)
    ar128 = jnp.arange(128)
    sa = (ar128[None, :] == (16 + ar8)[:, None]).astype(jnp.float32)
    sb = (ar128[None, :] == (24 + ar8)[:, None]).astype(jnp.float32)
    sc = (ar128[None, :] == 32).astype(jnp.float32).reshape(1, 128)
    wh1_p = jnp.pad(W_h1, ((0, 0), (0, 128 - 32)))
    bh1_p = jnp.pad(b_h1, (0, 128 - 32)).reshape(1, 128)
    wh2_p = jnp.pad(W_h2, ((0, 128 - 32), (0, 0)))
    tri = (jnp.arange(TBLK_A)[:, None] > jnp.arange(TBLK_A)[None, :]
           ).astype(jnp.float32)

    x, topi, topn, lr, g_rows, il = _prep_call(
        ints, opP, sa, sb, sc, w_in_p,
        b_in.reshape(1, D_MODEL), W_router, b_router.reshape(1, NUM_TILES),
        tri)
    p, plan = _plan_call(topi, lr, il)

    p_flat = p.reshape(NASSIGN)
    xs = _make_scatter()(x, p_flat)
    be = plan[:, 0]
    na = plan[0:1, 1].reshape(1)
    ys = _ffn_call(be, na, xs, W1, b1.reshape(NUM_TILES, 1, D_MODEL),
                   W2, b2.reshape(NUM_TILES, 1, D_MODEL))
    out = _make_combine()(ys, p_flat, g_rows.reshape(NASSIGN, 16))

    auxm = _aux_call(W1, W2, il)
    aux = auxm[0, 0]
    result_bits = _head_call(out, wh1_p, bh1_p, wh2_p, b_h2.reshape(1, 8))
    return result_bits, topi, aux
